# baseline (speedup 1.0000x reference)
import jax
import jax.numpy as jnp
from jax import lax
from jax.experimental import pallas as pl
from jax.experimental.pallas import tpu as pltpu
from jax.experimental.pallas import tpu_sc as plsc

N = 10000
E = 320000
P = 100000
F_IN = 128
H = 64

NC = 2
NS = 16
NW = NC * NS

N_PAD = 10240
E_BLOCKS = 80
E_PAD = NW * E_BLOCKS * 128
ET_BLOCKS = 160
P_BLOCKS = 25
P_PAD = NW * P_BLOCKS * 128

ROWS_PER_SUB = N_PAD // NS

SPLIT = 5120
ACC_ROWS = 5248
NBUF = 2


def _mesh():
    return plsc.VectorSubcoreMesh(core_axis_name="c", subcore_axis_name="s")


def _deg_body(dst_hbm, ones_hbm, zeros_hbm, out_hbm, didx, ones_v, acc):
    c = lax.axis_index("c")
    s = lax.axis_index("s")
    wid = s * NC + c
    r0 = s * ROWS_PER_SUB
    pltpu.sync_copy(zeros_hbm.at[pl.ds(r0, ROWS_PER_SUB)],
                    acc.at[pl.ds(r0, ROWS_PER_SUB)])
    pltpu.sync_copy(ones_hbm, ones_v)
    pltpu.sync_copy(dst_hbm.at[wid], didx)
    plsc.subcore_barrier()

    def body(j, carry):
        pltpu.sync_copy(ones_v, acc.at[didx.at[j]], add=True)
        return carry

    lax.fori_loop(0, E_BLOCKS, body, 0)
    plsc.subcore_barrier()
    pltpu.sync_copy(acc.at[pl.ds(r0, ROWS_PER_SUB)],
                    out_hbm.at[pl.ds(c * N_PAD + r0, ROWS_PER_SUB)])


_deg_kernel = pl.kernel(
    _deg_body,
    out_type=jax.ShapeDtypeStruct((NC * N_PAD, 2 * H), jnp.float32),
    mesh=_mesh(),
    scratch_types=[
        pltpu.VMEM((E_BLOCKS, 128), jnp.int32),
        pltpu.VMEM((128, 2 * H), jnp.float32),
        pltpu.VMEM_SHARED((N_PAD, 2 * H), jnp.float32),
    ],
)


def _edge_scatter_body(q_hbm, src_hbm, dstt_hbm, zeros_hbm, out_hbm,
                       sidx, didx, rows0, rows1, acc, sg0, sg1):
    c = lax.axis_index("c")
    s = lax.axis_index("s")
    zr = s * (ACC_ROWS // NS)
    pltpu.sync_copy(zeros_hbm.at[pl.ds(zr, ACC_ROWS // NS)],
                    acc.at[pl.ds(zr, ACC_ROWS // NS)])
    pltpu.sync_copy(src_hbm.at[s], sidx)
    pltpu.sync_copy(dstt_hbm.at[c * NS + s], didx)
    plsc.subcore_barrier()

    rows = (rows0, rows1)
    sg = (sg0, sg1)

    for k in range(NBUF):
        pltpu.async_copy(q_hbm.at[sidx.at[k]], rows[k], sg[k])

    def body(t4, carry):
        for k in range(NBUF):
            t = t4 * NBUF + k
            pltpu.make_async_copy(q_hbm.at[pl.ds(0, 128)], rows[k],
                                  sg[k]).wait()
            pltpu.sync_copy(rows[k], acc.at[didx.at[t]], add=True)

            @pl.when(t4 + 1 < ET_BLOCKS // NBUF)
            def _():
                pltpu.async_copy(q_hbm.at[sidx.at[t + NBUF]], rows[k], sg[k])
        return carry

    lax.fori_loop(0, ET_BLOCKS // NBUF, body, 0)
    plsc.subcore_barrier()
    wr = N_PAD // (2 * NS)
    @pl.when(c == 0)
    def _():
        pltpu.sync_copy(acc.at[pl.ds(s * wr, wr)],
                        out_hbm.at[pl.ds(s * wr, wr)])

    @pl.when(c == 1)
    def _():
        pltpu.sync_copy(acc.at[pl.ds(1 + s * wr, wr)],
                        out_hbm.at[pl.ds(SPLIT + s * wr, wr)])


_edge_scatter_kernel = pl.kernel(
    _edge_scatter_body,
    out_type=jax.ShapeDtypeStruct((N_PAD, 2 * H), jnp.float32),
    mesh=_mesh(),
    scratch_types=[
        pltpu.VMEM((ET_BLOCKS, 128), jnp.int32),
        pltpu.VMEM((ET_BLOCKS, 128), jnp.int32),
        pltpu.VMEM((128, 2 * H), jnp.float32),
        pltpu.VMEM((128, 2 * H), jnp.float32),
        pltpu.VMEM_SHARED((ACC_ROWS, 2 * H), jnp.float32),
        pltpu.SemaphoreType.DMA,
        pltpu.SemaphoreType.DMA,
    ],
)


FLOW_NBUF = 4
FB = 2 * P_BLOCKS + 2


def _flow_gather_body(ab_hbm, fidx_hbm, g_hbm,
                      fidx, rows0, rows1, rows2, rows3, sg0, sg1, sg2, sg3):
    c = lax.axis_index("c")
    s = lax.axis_index("s")
    wid = s * NC + c
    NB = FB
    pltpu.sync_copy(fidx_hbm.at[wid], fidx)

    rows = (rows0, rows1, rows2, rows3)
    sg = (sg0, sg1, sg2, sg3)
    for k in range(FLOW_NBUF):
        pltpu.async_copy(ab_hbm.at[fidx.at[k]], rows[k], sg[k])

    def body(t4, carry):
        for k in range(FLOW_NBUF):
            t = t4 * FLOW_NBUF + k
            pltpu.make_async_copy(ab_hbm.at[pl.ds(0, 128)], rows[k],
                                  sg[k]).wait()
            pltpu.sync_copy(rows[k],
                            g_hbm.at[pl.ds(wid * NB * 128 + t * 128, 128)])

            @pl.when(t4 + 1 < NB // FLOW_NBUF)
            def _():
                pltpu.async_copy(ab_hbm.at[fidx.at[t + FLOW_NBUF]], rows[k],
                                 sg[k])
        return carry

    lax.fori_loop(0, NB // FLOW_NBUF, body, 0)


_flow_gather_kernel = pl.kernel(
    _flow_gather_body,
    out_type=jax.ShapeDtypeStruct((NW * FB * 128, 2 * H), jnp.float32),
    mesh=_mesh(),
    scratch_types=[
        pltpu.VMEM((FB, 128), jnp.int32),
        pltpu.VMEM((128, 2 * H), jnp.float32),
        pltpu.VMEM((128, 2 * H), jnp.float32),
        pltpu.VMEM((128, 2 * H), jnp.float32),
        pltpu.VMEM((128, 2 * H), jnp.float32),
        pltpu.SemaphoreType.DMA,
        pltpu.SemaphoreType.DMA,
        pltpu.SemaphoreType.DMA,
        pltpu.SemaphoreType.DMA,
    ],
)


TC_BLK = 512
TC_GRID = N_PAD // TC_BLK


def _prep_body(dst_ref, dt_ref):
    d = dst_ref[...]
    dt_ref[0] = jnp.minimum(d, SPLIT)
    dt_ref[1] = jnp.maximum(d - (SPLIT - 1), 0)


def _tc_prep(dst2):
    return pl.pallas_call(
        _prep_body,
        grid=(8,),
        in_specs=[pl.BlockSpec((E_PAD // 128 // 8, 128), lambda i: (i, 0))],
        out_specs=pl.BlockSpec((NC, E_PAD // 128 // 8, 128),
                               lambda i: (0, i, 0)),
        out_shape=jax.ShapeDtypeStruct((NC, E_PAD // 128, 128), jnp.int32),
    )(dst2)


def _enc_body(x_ref, degp_ref, we_ref, be_ref, w1_ref, q1_ref, dinv_ref):
    cnt = degp_ref[0][:, 0:1] + degp_ref[1][:, 0:1]
    dinv = lax.rsqrt(cnt + 1.0)
    h0 = jax.nn.relu(
        jnp.dot(x_ref[...], we_ref[...], preferred_element_type=jnp.float32)
        + be_ref[...])
    q1 = jnp.dot(h0, w1_ref[...], preferred_element_type=jnp.float32) * dinv
    q1_ref[...] = jnp.concatenate(
        [q1, jnp.zeros((TC_BLK, H), jnp.float32)], axis=1)
    dinv_ref[...] = jnp.broadcast_to(dinv, (TC_BLK, 8))


def _tc_encoder(x_pad, degp, W_enc, b_enc, W1):
    return pl.pallas_call(
        _enc_body,
        grid=(TC_GRID,),
        in_specs=[
            pl.BlockSpec((TC_BLK, F_IN), lambda i: (i, 0)),
            pl.BlockSpec((NC, TC_BLK, 2 * H), lambda i: (0, i, 0)),
            pl.BlockSpec((F_IN, H), lambda i: (0, 0)),
            pl.BlockSpec((1, H), lambda i: (0, 0)),
            pl.BlockSpec((H, H), lambda i: (0, 0)),
        ],
        out_specs=[
            pl.BlockSpec((TC_BLK, 2 * H), lambda i: (i, 0)),
            pl.BlockSpec((TC_BLK, 8), lambda i: (i, 0)),
        ],
        out_shape=[
            jax.ShapeDtypeStruct((N_PAD, 2 * H), jnp.float32),
            jax.ShapeDtypeStruct((N_PAD, 8), jnp.float32),
        ],
    )(x_pad, degp, W_enc, b_enc.reshape(1, H), W1)


def _layer_body(s_ref, q_ref, dinv_ref, b_ref, wn_ref, qn_ref):
    dinv = dinv_ref[:, 0:1]
    h = jax.nn.relu((s_ref[:, :H] + q_ref[:, :H]) * dinv + b_ref[...])
    qn = jnp.dot(h, wn_ref[...], preferred_element_type=jnp.float32) * dinv
    qn_ref[...] = jnp.concatenate(
        [qn, jnp.zeros((TC_BLK, H), jnp.float32)], axis=1)


def _tc_layer(sp, q, dinv, b, W_next):
    return pl.pallas_call(
        _layer_body,
        grid=(TC_GRID,),
        in_specs=[
            pl.BlockSpec((TC_BLK, 2 * H), lambda i: (i, 0)),
            pl.BlockSpec((TC_BLK, 2 * H), lambda i: (i, 0)),
            pl.BlockSpec((TC_BLK, 8), lambda i: (i, 0)),
            pl.BlockSpec((1, H), lambda i: (0, 0)),
            pl.BlockSpec((H, H), lambda i: (0, 0)),
        ],
        out_specs=pl.BlockSpec((TC_BLK, 2 * H), lambda i: (i, 0)),
        out_shape=jax.ShapeDtypeStruct((N_PAD, 2 * H), jnp.float32),
    )(sp, q, dinv, b.reshape(1, H), W_next)


def _proj_body(s_ref, q_ref, dinv_ref, b_ref, wc_ref, ab_ref):
    dinv = dinv_ref[:, 0:1]
    h = jax.nn.relu((s_ref[:, :H] + q_ref[:, :H]) * dinv + b_ref[...])
    ab_ref[...] = jnp.dot(h, wc_ref[...], preferred_element_type=jnp.float32)


def _tc_proj(sp, q, dinv, b, Wf1):
    return pl.pallas_call(
        _proj_body,
        grid=(TC_GRID,),
        in_specs=[
            pl.BlockSpec((TC_BLK, 2 * H), lambda i: (i, 0)),
            pl.BlockSpec((TC_BLK, 2 * H), lambda i: (i, 0)),
            pl.BlockSpec((TC_BLK, 8), lambda i: (i, 0)),
            pl.BlockSpec((1, H), lambda i: (0, 0)),
            pl.BlockSpec((H, 2 * H), lambda i: (0, 0)),
        ],
        out_specs=pl.BlockSpec((TC_BLK, 2 * H), lambda i: (i, 0)),
        out_shape=jax.ShapeDtypeStruct((N_PAD, 2 * H), jnp.float32),
    )(sp, q, dinv, b.reshape(1, H),
      jnp.concatenate([Wf1[:H], Wf1[H:]], axis=1))


MLP_BLK = 1000
MLP_GRID = P // MLP_BLK


def _mlp_body(gs_ref, gt_ref, b1_ref, w2_ref, b2_ref, w3_ref, b3_ref, out_ref):
    z = jax.nn.relu(gs_ref[:, :H] + gt_ref[:, H:] + b1_ref[...])
    z2 = jax.nn.relu(
        jnp.dot(z, w2_ref[...], preferred_element_type=jnp.float32)
        + b2_ref[...])
    out_ref[...] = jnp.dot(z2, w3_ref[...],
                           preferred_element_type=jnp.float32) + b3_ref[...]


def _tc_mlp(gs, gt, bf1, Wf2, bf2, Wf3, bf3):
    return pl.pallas_call(
        _mlp_body,
        grid=(MLP_GRID,),
        in_specs=[
            pl.BlockSpec((MLP_BLK, 2 * H), lambda i: (i, 0)),
            pl.BlockSpec((MLP_BLK, 2 * H), lambda i: (i, 0)),
            pl.BlockSpec((1, H), lambda i: (0, 0)),
            pl.BlockSpec((H, H // 2), lambda i: (0, 0)),
            pl.BlockSpec((1, H // 2), lambda i: (0, 0)),
            pl.BlockSpec((H // 2, 1), lambda i: (0, 0)),
            pl.BlockSpec((1, 1), lambda i: (0, 0)),
        ],
        out_specs=pl.BlockSpec((MLP_BLK, 1), lambda i: (i, 0)),
        out_shape=jax.ShapeDtypeStruct((P, 1), jnp.float32),
    )(gs, gt, bf1.reshape(1, H), Wf2, bf2.reshape(1, H // 2), Wf3,
      bf3.reshape(1, 1))


def kernel(x, edge_index, flow_edges, W_enc, b_enc, W1, b1, W2, b2, W3, b3,
           Wf1, bf1, Wf2, bf2, Wf3, bf3):
    x_pad = jnp.concatenate(
        [x, jnp.zeros((N_PAD - N, F_IN), jnp.float32)], axis=0)

    def pad_flat(idx, total, fill):
        idx = idx.astype(jnp.int32)
        return jnp.concatenate(
            [idx, jnp.full((total - idx.shape[0],), fill, jnp.int32)])

    srcf = pad_flat(edge_index[0], E_PAD, N)
    dstf = pad_flat(edge_index[1], E_PAD, N)
    src16 = srcf.reshape(NS, ET_BLOCKS, 128)
    dst32 = dstf.reshape(NW, E_BLOCKS, 128)

    fsrc = pad_flat(flow_edges[0], P_PAD, 0).reshape(NW, P_BLOCKS, 128)
    ftgt = pad_flat(flow_edges[1], P_PAD, 0).reshape(NW, P_BLOCKS, 128)
    fpad = jnp.zeros((NW, FB - 2 * P_BLOCKS, 128), jnp.int32)
    fidx = jnp.concatenate([fsrc, ftgt, fpad], axis=1)

    zeros128 = jnp.zeros((N_PAD, 2 * H), jnp.float32)
    ones128 = jnp.ones((128, 2 * H), jnp.float32)

    dT = _tc_prep(dstf.reshape(E_PAD // 128, 128)).reshape(
        NC * NS, ET_BLOCKS, 128)

    degp = _deg_kernel(dst32, ones128, zeros128).reshape(NC, N_PAD, 2 * H)

    q1, dinv = _tc_encoder(x_pad, degp, W_enc, b_enc, W1)

    s1 = _edge_scatter_kernel(q1, src16, dT, zeros128)
    q2 = _tc_layer(s1, q1, dinv, b1, W2)
    s2 = _edge_scatter_kernel(q2, src16, dT, zeros128)
    q3 = _tc_layer(s2, q2, dinv, b2, W3)
    s3 = _edge_scatter_kernel(q3, src16, dT, zeros128)

    AB = _tc_proj(s3, q3, dinv, b3, Wf1)
    g = _flow_gather_kernel(AB, fidx)
    g = g.reshape(NW, FB * 128, 2 * H)
    gs = g[:, :P_BLOCKS * 128].reshape(P_PAD, 2 * H)
    gt = g[:, P_BLOCKS * 128:2 * P_BLOCKS * 128].reshape(P_PAD, 2 * H)
    flows = _tc_mlp(gs[:P], gt[:P], bf1, Wf2, bf2, Wf3, bf3)
    return flows


# speedup vs baseline: 1.2451x; 1.2451x over previous
import jax
import jax.numpy as jnp
from jax import lax
from jax.experimental import pallas as pl
from jax.experimental.pallas import tpu as pltpu
from jax.experimental.pallas import tpu_sc as plsc

N = 10000
E = 320000
P = 100000
F_IN = 128
H = 64

NC = 2
NS = 16
NW = NC * NS

N_PAD = 10240
E_BLOCKS = 80
E_PAD = NW * E_BLOCKS * 128
ET_BLOCKS = 160
P_BLOCKS = 25
P_PAD = NW * P_BLOCKS * 128

ROWS_PER_SUB = N_PAD // NS

SPLIT = 5120
ACC_ROWS = 5248
NBUF = 2


def _mesh():
    return plsc.VectorSubcoreMesh(core_axis_name="c", subcore_axis_name="s")


def _deg_body(dst_hbm, ones_hbm, zeros_hbm, out_hbm, didx, ones_v, acc):
    c = lax.axis_index("c")
    s = lax.axis_index("s")
    wid = s * NC + c
    r0 = s * ROWS_PER_SUB
    pltpu.sync_copy(zeros_hbm.at[pl.ds(r0, ROWS_PER_SUB)],
                    acc.at[pl.ds(r0, ROWS_PER_SUB)])
    pltpu.sync_copy(ones_hbm, ones_v)
    pltpu.sync_copy(dst_hbm.at[wid], didx)
    plsc.subcore_barrier()

    def body(j, carry):
        pltpu.sync_copy(ones_v, acc.at[didx.at[j]], add=True)
        return carry

    lax.fori_loop(0, E_BLOCKS, body, 0)
    plsc.subcore_barrier()
    pltpu.sync_copy(acc.at[pl.ds(r0, ROWS_PER_SUB)],
                    out_hbm.at[pl.ds(c * N_PAD + r0, ROWS_PER_SUB)])


_deg_kernel = pl.kernel(
    _deg_body,
    out_type=jax.ShapeDtypeStruct((NC * N_PAD, 2 * H), jnp.float32),
    mesh=_mesh(),
    scratch_types=[
        pltpu.VMEM((E_BLOCKS, 128), jnp.int32),
        pltpu.VMEM((128, 2 * H), jnp.float32),
        pltpu.VMEM_SHARED((N_PAD, 2 * H), jnp.float32),
    ],
)


def _edge_scatter_body(q_hbm, src_hbm, dst_hbm, zeros_hbm, out_hbm,
                       sidx, didx, rows, acc, sem):
    c = lax.axis_index("c")
    s = lax.axis_index("s")
    wid = s * NC + c
    r0 = s * ROWS_PER_SUB
    pltpu.sync_copy(zeros_hbm.at[pl.ds(r0, ROWS_PER_SUB)],
                    acc.at[pl.ds(r0, ROWS_PER_SUB)])
    pltpu.sync_copy(src_hbm.at[wid], sidx)
    pltpu.sync_copy(dst_hbm.at[wid], didx)
    plsc.subcore_barrier()

    def body(j, carry):
        pltpu.async_copy(q_hbm.at[sidx.at[j]], rows, sem).wait()
        pltpu.sync_copy(rows, acc.at[didx.at[j]], add=True)
        return carry

    lax.fori_loop(0, E_BLOCKS, body, 0)
    plsc.subcore_barrier()
    pltpu.sync_copy(acc.at[pl.ds(r0, ROWS_PER_SUB)],
                    out_hbm.at[pl.ds(c * N_PAD + r0, ROWS_PER_SUB)])


_edge_scatter_kernel = pl.kernel(
    _edge_scatter_body,
    out_type=jax.ShapeDtypeStruct((NC * N_PAD, 2 * H), jnp.float32),
    mesh=_mesh(),
    scratch_types=[
        pltpu.VMEM((E_BLOCKS, 128), jnp.int32),
        pltpu.VMEM((E_BLOCKS, 128), jnp.int32),
        pltpu.VMEM((128, 2 * H), jnp.float32),
        pltpu.VMEM_SHARED((N_PAD, 2 * H), jnp.float32),
        pltpu.SemaphoreType.DMA,
    ],
)


FB = 2 * P_BLOCKS + 2
FGB = 4


def _flow_gather_body(ab_hbm, fidx_hbm, g_hbm, fidx, rows, sem):
    c = lax.axis_index("c")
    s = lax.axis_index("s")
    wid = s * NC + c
    pltpu.sync_copy(fidx_hbm.at[wid], fidx)
    base = wid * FB * 128

    def body(t, carry):
        pltpu.async_copy(
            ab_hbm.at[fidx.at[pl.ds(t * FGB * 128, FGB * 128)]], rows,
            sem).wait()
        pltpu.sync_copy(rows,
                        g_hbm.at[pl.ds(base + t * FGB * 128, FGB * 128)])
        return carry

    lax.fori_loop(0, FB // FGB, body, 0)


_flow_gather_kernel = pl.kernel(
    _flow_gather_body,
    out_type=jax.ShapeDtypeStruct((NW * FB * 128, 2 * H), jnp.float32),
    mesh=_mesh(),
    scratch_types=[
        pltpu.VMEM((FB * 128,), jnp.int32),
        pltpu.VMEM((FGB * 128, 2 * H), jnp.float32),
        pltpu.SemaphoreType.DMA,
    ],
)


TC_BLK = 512
TC_GRID = N_PAD // TC_BLK


def _prep_body(dst_ref, dt_ref):
    d = dst_ref[...]
    dt_ref[0] = jnp.minimum(d, SPLIT)
    dt_ref[1] = jnp.maximum(d - (SPLIT - 1), 0)


def _tc_prep(dst2):
    return pl.pallas_call(
        _prep_body,
        grid=(8,),
        in_specs=[pl.BlockSpec((E_PAD // 128 // 8, 128), lambda i: (i, 0))],
        out_specs=pl.BlockSpec((NC, E_PAD // 128 // 8, 128),
                               lambda i: (0, i, 0)),
        out_shape=jax.ShapeDtypeStruct((NC, E_PAD // 128, 128), jnp.int32),
    )(dst2)


def _enc_body(x_ref, degp_ref, we_ref, be_ref, w1_ref, q1_ref, dinv_ref):
    cnt = degp_ref[0][:, 0:1] + degp_ref[1][:, 0:1]
    dinv = lax.rsqrt(cnt + 1.0)
    h0 = jax.nn.relu(
        jnp.dot(x_ref[...], we_ref[...], preferred_element_type=jnp.float32)
        + be_ref[...])
    q1 = jnp.dot(h0, w1_ref[...], preferred_element_type=jnp.float32) * dinv
    q1_ref[...] = jnp.concatenate(
        [q1, jnp.zeros((TC_BLK, H), jnp.float32)], axis=1)
    dinv_ref[...] = jnp.broadcast_to(dinv, (TC_BLK, 8))


def _tc_encoder(x_pad, degp, W_enc, b_enc, W1):
    return pl.pallas_call(
        _enc_body,
        grid=(TC_GRID,),
        in_specs=[
            pl.BlockSpec((TC_BLK, F_IN), lambda i: (i, 0)),
            pl.BlockSpec((NC, TC_BLK, 2 * H), lambda i: (0, i, 0)),
            pl.BlockSpec((F_IN, H), lambda i: (0, 0)),
            pl.BlockSpec((1, H), lambda i: (0, 0)),
            pl.BlockSpec((H, H), lambda i: (0, 0)),
        ],
        out_specs=[
            pl.BlockSpec((TC_BLK, 2 * H), lambda i: (i, 0)),
            pl.BlockSpec((TC_BLK, 8), lambda i: (i, 0)),
        ],
        out_shape=[
            jax.ShapeDtypeStruct((N_PAD, 2 * H), jnp.float32),
            jax.ShapeDtypeStruct((N_PAD, 8), jnp.float32),
        ],
    )(x_pad, degp, W_enc, b_enc.reshape(1, H), W1)


def _layer_body(s_ref, q_ref, dinv_ref, b_ref, wn_ref, qn_ref):
    dinv = dinv_ref[:, 0:1]
    h = jax.nn.relu(
        (s_ref[0][:, :H] + s_ref[1][:, :H] + q_ref[:, :H]) * dinv
        + b_ref[...])
    qn = jnp.dot(h, wn_ref[...], preferred_element_type=jnp.float32) * dinv
    qn_ref[...] = jnp.concatenate(
        [qn, jnp.zeros((TC_BLK, H), jnp.float32)], axis=1)


def _tc_layer(sp, q, dinv, b, W_next):
    return pl.pallas_call(
        _layer_body,
        grid=(TC_GRID,),
        in_specs=[
            pl.BlockSpec((NC, TC_BLK, 2 * H), lambda i: (0, i, 0)),
            pl.BlockSpec((TC_BLK, 2 * H), lambda i: (i, 0)),
            pl.BlockSpec((TC_BLK, 8), lambda i: (i, 0)),
            pl.BlockSpec((1, H), lambda i: (0, 0)),
            pl.BlockSpec((H, H), lambda i: (0, 0)),
        ],
        out_specs=pl.BlockSpec((TC_BLK, 2 * H), lambda i: (i, 0)),
        out_shape=jax.ShapeDtypeStruct((N_PAD, 2 * H), jnp.float32),
    )(sp, q, dinv, b.reshape(1, H), W_next)


def _proj_body(s_ref, q_ref, dinv_ref, b_ref, wc_ref, ab_ref):
    dinv = dinv_ref[:, 0:1]
    h = jax.nn.relu(
        (s_ref[0][:, :H] + s_ref[1][:, :H] + q_ref[:, :H]) * dinv
        + b_ref[...])
    ab_ref[...] = jnp.dot(h, wc_ref[...], preferred_element_type=jnp.float32)


def _tc_proj(sp, q, dinv, b, Wf1):
    return pl.pallas_call(
        _proj_body,
        grid=(TC_GRID,),
        in_specs=[
            pl.BlockSpec((NC, TC_BLK, 2 * H), lambda i: (0, i, 0)),
            pl.BlockSpec((TC_BLK, 2 * H), lambda i: (i, 0)),
            pl.BlockSpec((TC_BLK, 8), lambda i: (i, 0)),
            pl.BlockSpec((1, H), lambda i: (0, 0)),
            pl.BlockSpec((H, 2 * H), lambda i: (0, 0)),
        ],
        out_specs=pl.BlockSpec((TC_BLK, 2 * H), lambda i: (i, 0)),
        out_shape=jax.ShapeDtypeStruct((N_PAD, 2 * H), jnp.float32),
    )(sp, q, dinv, b.reshape(1, H),
      jnp.concatenate([Wf1[:H], Wf1[H:]], axis=1))


MLP_BLK = 1000
MLP_GRID = P // MLP_BLK


def _mlp_body(gs_ref, gt_ref, b1_ref, w2_ref, b2_ref, w3_ref, b3_ref, out_ref):
    z = jax.nn.relu(gs_ref[:, :H] + gt_ref[:, H:] + b1_ref[...])
    z2 = jax.nn.relu(
        jnp.dot(z, w2_ref[...], preferred_element_type=jnp.float32)
        + b2_ref[...])
    out_ref[...] = jnp.dot(z2, w3_ref[...],
                           preferred_element_type=jnp.float32) + b3_ref[...]


def _tc_mlp(gs, gt, bf1, Wf2, bf2, Wf3, bf3):
    return pl.pallas_call(
        _mlp_body,
        grid=(MLP_GRID,),
        in_specs=[
            pl.BlockSpec((MLP_BLK, 2 * H), lambda i: (i, 0)),
            pl.BlockSpec((MLP_BLK, 2 * H), lambda i: (i, 0)),
            pl.BlockSpec((1, H), lambda i: (0, 0)),
            pl.BlockSpec((H, H // 2), lambda i: (0, 0)),
            pl.BlockSpec((1, H // 2), lambda i: (0, 0)),
            pl.BlockSpec((H // 2, 1), lambda i: (0, 0)),
            pl.BlockSpec((1, 1), lambda i: (0, 0)),
        ],
        out_specs=pl.BlockSpec((MLP_BLK, 1), lambda i: (i, 0)),
        out_shape=jax.ShapeDtypeStruct((P, 1), jnp.float32),
    )(gs, gt, bf1.reshape(1, H), Wf2, bf2.reshape(1, H // 2), Wf3,
      bf3.reshape(1, 1))


def kernel(x, edge_index, flow_edges, W_enc, b_enc, W1, b1, W2, b2, W3, b3,
           Wf1, bf1, Wf2, bf2, Wf3, bf3):
    x_pad = jnp.concatenate(
        [x, jnp.zeros((N_PAD - N, F_IN), jnp.float32)], axis=0)

    def pad_flat(idx, total, fill):
        idx = idx.astype(jnp.int32)
        return jnp.concatenate(
            [idx, jnp.full((total - idx.shape[0],), fill, jnp.int32)])

    srcf = pad_flat(edge_index[0], E_PAD, N)
    dstf = pad_flat(edge_index[1], E_PAD, N)
    src32 = srcf.reshape(NW, E_BLOCKS, 128)
    dst32 = dstf.reshape(NW, E_BLOCKS, 128)

    fsrc = pad_flat(flow_edges[0], P_PAD, 0).reshape(NW, P_BLOCKS, 128)
    ftgt = pad_flat(flow_edges[1], P_PAD, 0).reshape(NW, P_BLOCKS, 128)
    fpad = jnp.zeros((NW, FB - 2 * P_BLOCKS, 128), jnp.int32)
    fidx = jnp.concatenate([fsrc, ftgt, fpad], axis=1).reshape(NW, FB * 128)

    zeros128 = jnp.zeros((N_PAD, 2 * H), jnp.float32)
    ones128 = jnp.ones((128, 2 * H), jnp.float32)

    degp = _deg_kernel(dst32, ones128, zeros128).reshape(NC, N_PAD, 2 * H)

    q1, dinv = _tc_encoder(x_pad, degp, W_enc, b_enc, W1)

    s1 = _edge_scatter_kernel(q1, src32, dst32,
                              zeros128).reshape(NC, N_PAD, 2 * H)
    q2 = _tc_layer(s1, q1, dinv, b1, W2)
    s2 = _edge_scatter_kernel(q2, src32, dst32,
                              zeros128).reshape(NC, N_PAD, 2 * H)
    q3 = _tc_layer(s2, q2, dinv, b2, W3)
    s3 = _edge_scatter_kernel(q3, src32, dst32,
                              zeros128).reshape(NC, N_PAD, 2 * H)

    AB = _tc_proj(s3, q3, dinv, b3, Wf1)
    g = _flow_gather_kernel(AB, fidx)
    g = g.reshape(NW, FB * 128, 2 * H)
    gs = g[:, :P_BLOCKS * 128].reshape(P_PAD, 2 * H)
    gt = g[:, P_BLOCKS * 128:2 * P_BLOCKS * 128].reshape(P_PAD, 2 * H)
    flows = _tc_mlp(gs[:P], gt[:P], bf1, Wf2, bf2, Wf3, bf3)
    return flows


# speedup vs baseline: 1.3574x; 1.0902x over previous
import jax
import jax.numpy as jnp
from jax import lax
from jax.experimental import pallas as pl
from jax.experimental.pallas import tpu as pltpu
from jax.experimental.pallas import tpu_sc as plsc

N = 10000
E = 320000
P = 100000
F_IN = 128
H = 64

NC = 2
NS = 16
NW = NC * NS

N_PAD = 10240
E_BLOCKS = 80
E_PAD = NW * E_BLOCKS * 128
ET_BLOCKS = 160
P_BLOCKS = 25
P_PAD = NW * P_BLOCKS * 128

ROWS_PER_SUB = N_PAD // NS

SPLIT = 5120
ACC_ROWS = 5248
NBUF = 2


def _mesh():
    return plsc.VectorSubcoreMesh(core_axis_name="c", subcore_axis_name="s")


def _deg_body(dst_hbm, ones_hbm, zeros_hbm, out_hbm, didx, ones_v, acc):
    c = lax.axis_index("c")
    s = lax.axis_index("s")
    wid = s * NC + c
    r0 = s * ROWS_PER_SUB
    pltpu.sync_copy(zeros_hbm.at[pl.ds(r0, ROWS_PER_SUB)],
                    acc.at[pl.ds(r0, ROWS_PER_SUB)])
    pltpu.sync_copy(ones_hbm, ones_v)
    pltpu.sync_copy(dst_hbm.at[wid], didx)
    plsc.subcore_barrier()

    def body(j, carry):
        pltpu.sync_copy(ones_v, acc.at[didx.at[j]], add=True)
        return carry

    lax.fori_loop(0, E_BLOCKS, body, 0)
    plsc.subcore_barrier()
    pltpu.sync_copy(acc.at[pl.ds(r0, ROWS_PER_SUB)],
                    out_hbm.at[pl.ds(c * N_PAD + r0, ROWS_PER_SUB)])


_deg_kernel = pl.kernel(
    _deg_body,
    out_type=jax.ShapeDtypeStruct((NC * N_PAD, 2 * H), jnp.float32),
    mesh=_mesh(),
    scratch_types=[
        pltpu.VMEM((E_BLOCKS, 128), jnp.int32),
        pltpu.VMEM((128, 2 * H), jnp.float32),
        pltpu.VMEM_SHARED((N_PAD, 2 * H), jnp.float32),
    ],
)


NB0 = 40
NB1 = 120


def _edge_scatter_body(q_hbm, src_hbm, dst_hbm, zeros_hbm, out_hbm,
                       sidx, didx, rows, acc, sem):
    c = lax.axis_index("c")
    s = lax.axis_index("s")
    r0 = s * ROWS_PER_SUB
    pltpu.sync_copy(zeros_hbm.at[pl.ds(r0, ROWS_PER_SUB)],
                    acc.at[pl.ds(r0, ROWS_PER_SUB)])

    @pl.when(c == 0)
    def _():
        pltpu.sync_copy(src_hbm.at[pl.ds(s * NB0, NB0)],
                        sidx.at[pl.ds(0, NB0)])
        pltpu.sync_copy(dst_hbm.at[pl.ds(s * NB0, NB0)],
                        didx.at[pl.ds(0, NB0)])

    @pl.when(c == 1)
    def _():
        pltpu.sync_copy(src_hbm.at[pl.ds(NS * NB0 + s * NB1, NB1)],
                        sidx.at[pl.ds(0, NB1)])
        pltpu.sync_copy(dst_hbm.at[pl.ds(NS * NB0 + s * NB1, NB1)],
                        didx.at[pl.ds(0, NB1)])

    plsc.subcore_barrier()
    nb = jnp.where(c == 0, NB0, NB1)

    def body(j, carry):
        pltpu.async_copy(q_hbm.at[sidx.at[j]], rows, sem).wait()
        pltpu.sync_copy(rows, acc.at[didx.at[j]], add=True)
        return carry

    lax.fori_loop(0, nb, body, 0)
    plsc.subcore_barrier()
    pltpu.sync_copy(acc.at[pl.ds(r0, ROWS_PER_SUB)],
                    out_hbm.at[pl.ds(c * N_PAD + r0, ROWS_PER_SUB)])


_edge_scatter_kernel = pl.kernel(
    _edge_scatter_body,
    out_type=jax.ShapeDtypeStruct((NC * N_PAD, 2 * H), jnp.float32),
    mesh=_mesh(),
    scratch_types=[
        pltpu.VMEM((NB1, 128), jnp.int32),
        pltpu.VMEM((NB1, 128), jnp.int32),
        pltpu.VMEM((128, 2 * H), jnp.float32),
        pltpu.VMEM_SHARED((N_PAD, 2 * H), jnp.float32),
        pltpu.SemaphoreType.DMA,
    ],
)


FB = 2 * P_BLOCKS


def _flow_gather_body(ab_hbm, fidx_hbm, g_hbm, fidx, rows_a, rows_b,
                      sem_a, sem_b):
    c = lax.axis_index("c")
    s = lax.axis_index("s")
    wid = s * NC + c
    pltpu.sync_copy(fidx_hbm.at[wid], fidx)
    base = wid * FB * 128

    def body(t, carry):
        da = pltpu.async_copy(ab_hbm.at[fidx.at[t]], rows_a, sem_a)
        db = pltpu.async_copy(ab_hbm.at[fidx.at[P_BLOCKS + t]], rows_b,
                              sem_b)
        da.wait()
        pltpu.sync_copy(rows_a, g_hbm.at[pl.ds(base + t * 128, 128)])
        db.wait()
        pltpu.sync_copy(
            rows_b,
            g_hbm.at[pl.ds(base + (P_BLOCKS + t) * 128, 128)])
        return carry

    lax.fori_loop(0, P_BLOCKS, body, 0)


_flow_gather_kernel = pl.kernel(
    _flow_gather_body,
    out_type=jax.ShapeDtypeStruct((NW * FB * 128, 2 * H), jnp.float32),
    mesh=_mesh(),
    scratch_types=[
        pltpu.VMEM((FB, 128), jnp.int32),
        pltpu.VMEM((128, 2 * H), jnp.float32),
        pltpu.VMEM((128, 2 * H), jnp.float32),
        pltpu.SemaphoreType.DMA,
        pltpu.SemaphoreType.DMA,
    ],
)


TC_BLK = 512
TC_GRID = N_PAD // TC_BLK


def _prep_body(dst_ref, dt_ref):
    d = dst_ref[...]
    dt_ref[0] = jnp.minimum(d, SPLIT)
    dt_ref[1] = jnp.maximum(d - (SPLIT - 1), 0)


def _tc_prep(dst2):
    return pl.pallas_call(
        _prep_body,
        grid=(8,),
        in_specs=[pl.BlockSpec((E_PAD // 128 // 8, 128), lambda i: (i, 0))],
        out_specs=pl.BlockSpec((NC, E_PAD // 128 // 8, 128),
                               lambda i: (0, i, 0)),
        out_shape=jax.ShapeDtypeStruct((NC, E_PAD // 128, 128), jnp.int32),
    )(dst2)


def _enc_body(x_ref, degp_ref, we_ref, be_ref, w1_ref, q1_ref, dinv_ref):
    cnt = degp_ref[0][:, 0:1] + degp_ref[1][:, 0:1]
    dinv = lax.rsqrt(cnt + 1.0)
    h0 = jax.nn.relu(
        jnp.dot(x_ref[...], we_ref[...], preferred_element_type=jnp.float32)
        + be_ref[...])
    q1 = jnp.dot(h0, w1_ref[...], preferred_element_type=jnp.float32) * dinv
    q1_ref[...] = jnp.concatenate(
        [q1, jnp.zeros((TC_BLK, H), jnp.float32)], axis=1)
    dinv_ref[...] = jnp.broadcast_to(dinv, (TC_BLK, 8))


def _tc_encoder(x_pad, degp, W_enc, b_enc, W1):
    return pl.pallas_call(
        _enc_body,
        grid=(TC_GRID,),
        in_specs=[
            pl.BlockSpec((TC_BLK, F_IN), lambda i: (i, 0)),
            pl.BlockSpec((NC, TC_BLK, 2 * H), lambda i: (0, i, 0)),
            pl.BlockSpec((F_IN, H), lambda i: (0, 0)),
            pl.BlockSpec((1, H), lambda i: (0, 0)),
            pl.BlockSpec((H, H), lambda i: (0, 0)),
        ],
        out_specs=[
            pl.BlockSpec((TC_BLK, 2 * H), lambda i: (i, 0)),
            pl.BlockSpec((TC_BLK, 8), lambda i: (i, 0)),
        ],
        out_shape=[
            jax.ShapeDtypeStruct((N_PAD, 2 * H), jnp.float32),
            jax.ShapeDtypeStruct((N_PAD, 8), jnp.float32),
        ],
    )(x_pad, degp, W_enc, b_enc.reshape(1, H), W1)


def _layer_body(s_ref, q_ref, dinv_ref, b_ref, wn_ref, qn_ref):
    dinv = dinv_ref[:, 0:1]
    h = jax.nn.relu(
        (s_ref[0][:, :H] + s_ref[1][:, :H] + q_ref[:, :H]) * dinv
        + b_ref[...])
    qn = jnp.dot(h, wn_ref[...], preferred_element_type=jnp.float32) * dinv
    qn_ref[...] = jnp.concatenate(
        [qn, jnp.zeros((TC_BLK, H), jnp.float32)], axis=1)


def _tc_layer(sp, q, dinv, b, W_next):
    return pl.pallas_call(
        _layer_body,
        grid=(TC_GRID,),
        in_specs=[
            pl.BlockSpec((NC, TC_BLK, 2 * H), lambda i: (0, i, 0)),
            pl.BlockSpec((TC_BLK, 2 * H), lambda i: (i, 0)),
            pl.BlockSpec((TC_BLK, 8), lambda i: (i, 0)),
            pl.BlockSpec((1, H), lambda i: (0, 0)),
            pl.BlockSpec((H, H), lambda i: (0, 0)),
        ],
        out_specs=pl.BlockSpec((TC_BLK, 2 * H), lambda i: (i, 0)),
        out_shape=jax.ShapeDtypeStruct((N_PAD, 2 * H), jnp.float32),
    )(sp, q, dinv, b.reshape(1, H), W_next)


def _proj_body(s_ref, q_ref, dinv_ref, b_ref, wc_ref, ab_ref):
    dinv = dinv_ref[:, 0:1]
    h = jax.nn.relu(
        (s_ref[0][:, :H] + s_ref[1][:, :H] + q_ref[:, :H]) * dinv
        + b_ref[...])
    ab_ref[...] = jnp.dot(h, wc_ref[...], preferred_element_type=jnp.float32)


def _tc_proj(sp, q, dinv, b, Wf1):
    return pl.pallas_call(
        _proj_body,
        grid=(TC_GRID,),
        in_specs=[
            pl.BlockSpec((NC, TC_BLK, 2 * H), lambda i: (0, i, 0)),
            pl.BlockSpec((TC_BLK, 2 * H), lambda i: (i, 0)),
            pl.BlockSpec((TC_BLK, 8), lambda i: (i, 0)),
            pl.BlockSpec((1, H), lambda i: (0, 0)),
            pl.BlockSpec((H, 2 * H), lambda i: (0, 0)),
        ],
        out_specs=pl.BlockSpec((TC_BLK, 2 * H), lambda i: (i, 0)),
        out_shape=jax.ShapeDtypeStruct((N_PAD, 2 * H), jnp.float32),
    )(sp, q, dinv, b.reshape(1, H),
      jnp.concatenate([Wf1[:H], Wf1[H:]], axis=1))


MLP_BLK = 1000
MLP_GRID = P // MLP_BLK


def _mlp_body(gs_ref, gt_ref, b1_ref, w2_ref, b2_ref, w3_ref, b3_ref, out_ref):
    z = jax.nn.relu(gs_ref[:, :H] + gt_ref[:, H:] + b1_ref[...])
    z2 = jax.nn.relu(
        jnp.dot(z, w2_ref[...], preferred_element_type=jnp.float32)
        + b2_ref[...])
    out_ref[...] = jnp.dot(z2, w3_ref[...],
                           preferred_element_type=jnp.float32) + b3_ref[...]


def _tc_mlp(gs, gt, bf1, Wf2, bf2, Wf3, bf3):
    return pl.pallas_call(
        _mlp_body,
        grid=(MLP_GRID,),
        in_specs=[
            pl.BlockSpec((MLP_BLK, 2 * H), lambda i: (i, 0)),
            pl.BlockSpec((MLP_BLK, 2 * H), lambda i: (i, 0)),
            pl.BlockSpec((1, H), lambda i: (0, 0)),
            pl.BlockSpec((H, H // 2), lambda i: (0, 0)),
            pl.BlockSpec((1, H // 2), lambda i: (0, 0)),
            pl.BlockSpec((H // 2, 1), lambda i: (0, 0)),
            pl.BlockSpec((1, 1), lambda i: (0, 0)),
        ],
        out_specs=pl.BlockSpec((MLP_BLK, 1), lambda i: (i, 0)),
        out_shape=jax.ShapeDtypeStruct((P, 1), jnp.float32),
    )(gs, gt, bf1.reshape(1, H), Wf2, bf2.reshape(1, H // 2), Wf3,
      bf3.reshape(1, 1))


def kernel(x, edge_index, flow_edges, W_enc, b_enc, W1, b1, W2, b2, W3, b3,
           Wf1, bf1, Wf2, bf2, Wf3, bf3):
    x_pad = jnp.concatenate(
        [x, jnp.zeros((N_PAD - N, F_IN), jnp.float32)], axis=0)

    def pad_flat(idx, total, fill):
        idx = idx.astype(jnp.int32)
        return jnp.concatenate(
            [idx, jnp.full((total - idx.shape[0],), fill, jnp.int32)])

    srcf = pad_flat(edge_index[0], E_PAD, N)
    dstf = pad_flat(edge_index[1], E_PAD, N)
    src32 = srcf.reshape(E_PAD // 128, 128)
    dst32 = dstf.reshape(E_PAD // 128, 128)
    dst32w = dstf.reshape(NW, E_BLOCKS, 128)

    fsrc = pad_flat(flow_edges[0], P_PAD, 0).reshape(NW, P_BLOCKS, 128)
    ftgt = pad_flat(flow_edges[1], P_PAD, 0).reshape(NW, P_BLOCKS, 128)
    fidx = jnp.concatenate([fsrc, ftgt], axis=1)

    zeros128 = jnp.zeros((N_PAD, 2 * H), jnp.float32)
    ones128 = jnp.ones((128, 2 * H), jnp.float32)

    degp = _deg_kernel(dst32w, ones128, zeros128).reshape(NC, N_PAD, 2 * H)

    q1, dinv = _tc_encoder(x_pad, degp, W_enc, b_enc, W1)

    s1 = _edge_scatter_kernel(q1, src32, dst32,
                              zeros128).reshape(NC, N_PAD, 2 * H)
    q2 = _tc_layer(s1, q1, dinv, b1, W2)
    s2 = _edge_scatter_kernel(q2, src32, dst32,
                              zeros128).reshape(NC, N_PAD, 2 * H)
    q3 = _tc_layer(s2, q2, dinv, b2, W3)
    s3 = _edge_scatter_kernel(q3, src32, dst32,
                              zeros128).reshape(NC, N_PAD, 2 * H)

    AB = _tc_proj(s3, q3, dinv, b3, Wf1)
    g = _flow_gather_kernel(AB, fidx)
    g = g.reshape(NW, FB * 128, 2 * H)
    gs = g[:, :P_BLOCKS * 128].reshape(P_PAD, 2 * H)
    gt = g[:, P_BLOCKS * 128:2 * P_BLOCKS * 128].reshape(P_PAD, 2 * H)
    flows = _tc_mlp(gs[:P], gt[:P], bf1, Wf2, bf2, Wf3, bf3)
    return flows


# speedup vs baseline: 1.6705x; 1.2307x over previous
import jax
import jax.numpy as jnp
from jax import lax
from jax.experimental import pallas as pl
from jax.experimental.pallas import tpu as pltpu
from jax.experimental.pallas import tpu_sc as plsc

N = 10000
E = 320000
P = 100000
F_IN = 128
H = 64

NC = 2
NS = 16
NW = NC * NS

N_PAD = 10240
E_BLOCKS = 80
E_PAD = NW * E_BLOCKS * 128
ET_BLOCKS = 160
P_BLOCKS = 25
P_PAD = NW * P_BLOCKS * 128

ROWS_PER_SUB = N_PAD // NS

SPLIT = 5120
ACC_ROWS = 5248
NBUF = 2


def _mesh():
    return plsc.VectorSubcoreMesh(core_axis_name="c", subcore_axis_name="s")


def _deg_body(dst_hbm, ones_hbm, zeros_hbm, out_hbm, didx, ones_v, acc):
    c = lax.axis_index("c")
    s = lax.axis_index("s")
    wid = s * NC + c
    r0 = s * ROWS_PER_SUB
    pltpu.sync_copy(zeros_hbm.at[pl.ds(r0, ROWS_PER_SUB)],
                    acc.at[pl.ds(r0, ROWS_PER_SUB)])
    pltpu.sync_copy(ones_hbm, ones_v)
    pltpu.sync_copy(dst_hbm.at[wid], didx)
    plsc.subcore_barrier()

    def body(j, carry):
        pltpu.sync_copy(ones_v, acc.at[didx.at[j]], add=True)
        return carry

    lax.fori_loop(0, E_BLOCKS, body, 0)
    plsc.subcore_barrier()
    pltpu.sync_copy(acc.at[pl.ds(r0, ROWS_PER_SUB)],
                    out_hbm.at[pl.ds(c * N_PAD + r0, ROWS_PER_SUB)])


_deg_kernel = pl.kernel(
    _deg_body,
    out_type=jax.ShapeDtypeStruct((NC * N_PAD, 2 * H), jnp.float32),
    mesh=_mesh(),
    scratch_types=[
        pltpu.VMEM((E_BLOCKS, 128), jnp.int32),
        pltpu.VMEM((128, 2 * H), jnp.float32),
        pltpu.VMEM_SHARED((N_PAD, 2 * H), jnp.float32),
    ],
)


NB0 = 120
NB1 = 40


def _edge_scatter_body(q_hbm, src_hbm, dst_hbm, zeros_hbm, out_hbm,
                       sidx, didx, rows, acc, sem):
    c = lax.axis_index("c")
    s = lax.axis_index("s")
    r0 = s * ROWS_PER_SUB
    pltpu.sync_copy(zeros_hbm.at[pl.ds(r0, ROWS_PER_SUB)],
                    acc.at[pl.ds(r0, ROWS_PER_SUB)])

    @pl.when(c == 0)
    def _():
        pltpu.sync_copy(src_hbm.at[pl.ds(s * NB0, NB0)],
                        sidx.at[pl.ds(0, NB0)])
        pltpu.sync_copy(dst_hbm.at[pl.ds(s * NB0, NB0)],
                        didx.at[pl.ds(0, NB0)])

    @pl.when(c == 1)
    def _():
        pltpu.sync_copy(src_hbm.at[pl.ds(NS * NB0 + s * NB1, NB1)],
                        sidx.at[pl.ds(0, NB1)])
        pltpu.sync_copy(dst_hbm.at[pl.ds(NS * NB0 + s * NB1, NB1)],
                        didx.at[pl.ds(0, NB1)])

    plsc.subcore_barrier()
    nb = jnp.where(c == 0, NB0, NB1)

    def body(j, carry):
        pltpu.async_copy(q_hbm.at[sidx.at[j]], rows, sem).wait()
        pltpu.sync_copy(rows, acc.at[didx.at[j]], add=True)
        return carry

    lax.fori_loop(0, nb, body, 0)
    plsc.subcore_barrier()
    pltpu.sync_copy(acc.at[pl.ds(r0, ROWS_PER_SUB)],
                    out_hbm.at[pl.ds(c * N_PAD + r0, ROWS_PER_SUB)])


_edge_scatter_kernel = pl.kernel(
    _edge_scatter_body,
    out_type=jax.ShapeDtypeStruct((NC * N_PAD, 2 * H), jnp.float32),
    mesh=_mesh(),
    scratch_types=[
        pltpu.VMEM((max(NB0, NB1), 128), jnp.int32),
        pltpu.VMEM((max(NB0, NB1), 128), jnp.int32),
        pltpu.VMEM((128, 2 * H), jnp.float32),
        pltpu.VMEM_SHARED((N_PAD, 2 * H), jnp.float32),
        pltpu.SemaphoreType.DMA,
    ],
)


FB = 2 * P_BLOCKS
NF0 = 80
NF1 = 20


def _flow_gather_body(ab_hbm, fa_hbm, fb_hbm, g_hbm, fidx, rows, sem):
    c = lax.axis_index("c")
    s = lax.axis_index("s")

    @pl.when(c == 0)
    def _():
        pltpu.sync_copy(fa_hbm.at[s], fidx.at[pl.ds(0, NF0)])

    @pl.when(c == 1)
    def _():
        pltpu.sync_copy(fb_hbm.at[s], fidx.at[pl.ds(0, NF1)])

    nf = jnp.where(c == 0, NF0, NF1)
    base = jnp.where(c == 0, s * NF0, NS * NF0 + s * NF1) * 128

    def body(t, carry):
        pltpu.async_copy(ab_hbm.at[fidx.at[t]], rows, sem).wait()
        pltpu.sync_copy(rows, g_hbm.at[pl.ds(base + t * 128, 128)])
        return carry

    lax.fori_loop(0, nf, body, 0)


_flow_gather_kernel = pl.kernel(
    _flow_gather_body,
    out_type=jax.ShapeDtypeStruct((NW * FB * 128, 2 * H), jnp.float32),
    mesh=_mesh(),
    scratch_types=[
        pltpu.VMEM((NF0, 128), jnp.int32),
        pltpu.VMEM((128, 2 * H), jnp.float32),
        pltpu.SemaphoreType.DMA,
    ],
)


TC_BLK = 512
TC_GRID = N_PAD // TC_BLK


def _prep_body(dst_ref, dt_ref):
    d = dst_ref[...]
    dt_ref[0] = jnp.minimum(d, SPLIT)
    dt_ref[1] = jnp.maximum(d - (SPLIT - 1), 0)


def _tc_prep(dst2):
    return pl.pallas_call(
        _prep_body,
        grid=(8,),
        in_specs=[pl.BlockSpec((E_PAD // 128 // 8, 128), lambda i: (i, 0))],
        out_specs=pl.BlockSpec((NC, E_PAD // 128 // 8, 128),
                               lambda i: (0, i, 0)),
        out_shape=jax.ShapeDtypeStruct((NC, E_PAD // 128, 128), jnp.int32),
    )(dst2)


def _enc_body(x_ref, degp_ref, we_ref, be_ref, w1_ref, q1_ref, dinv_ref):
    cnt = degp_ref[0][:, 0:1] + degp_ref[1][:, 0:1]
    dinv = lax.rsqrt(cnt + 1.0)
    h0 = jax.nn.relu(
        jnp.dot(x_ref[...], we_ref[...], preferred_element_type=jnp.float32)
        + be_ref[...])
    q1 = jnp.dot(h0, w1_ref[...], preferred_element_type=jnp.float32) * dinv
    q1_ref[...] = jnp.concatenate(
        [q1, jnp.zeros((TC_BLK, H), jnp.float32)], axis=1)
    dinv_ref[...] = jnp.broadcast_to(dinv, (TC_BLK, 8))


def _tc_encoder(x_pad, degp, W_enc, b_enc, W1):
    return pl.pallas_call(
        _enc_body,
        grid=(TC_GRID,),
        in_specs=[
            pl.BlockSpec((TC_BLK, F_IN), lambda i: (i, 0)),
            pl.BlockSpec((NC, TC_BLK, 2 * H), lambda i: (0, i, 0)),
            pl.BlockSpec((F_IN, H), lambda i: (0, 0)),
            pl.BlockSpec((1, H), lambda i: (0, 0)),
            pl.BlockSpec((H, H), lambda i: (0, 0)),
        ],
        out_specs=[
            pl.BlockSpec((TC_BLK, 2 * H), lambda i: (i, 0)),
            pl.BlockSpec((TC_BLK, 8), lambda i: (i, 0)),
        ],
        out_shape=[
            jax.ShapeDtypeStruct((N_PAD, 2 * H), jnp.float32),
            jax.ShapeDtypeStruct((N_PAD, 8), jnp.float32),
        ],
    )(x_pad, degp, W_enc, b_enc.reshape(1, H), W1)


def _layer_body(s_ref, q_ref, dinv_ref, b_ref, wn_ref, qn_ref):
    dinv = dinv_ref[:, 0:1]
    h = jax.nn.relu(
        (s_ref[0][:, :H] + s_ref[1][:, :H] + q_ref[:, :H]) * dinv
        + b_ref[...])
    qn = jnp.dot(h, wn_ref[...], preferred_element_type=jnp.float32) * dinv
    qn_ref[...] = jnp.concatenate(
        [qn, jnp.zeros((TC_BLK, H), jnp.float32)], axis=1)


def _tc_layer(sp, q, dinv, b, W_next):
    return pl.pallas_call(
        _layer_body,
        grid=(TC_GRID,),
        in_specs=[
            pl.BlockSpec((NC, TC_BLK, 2 * H), lambda i: (0, i, 0)),
            pl.BlockSpec((TC_BLK, 2 * H), lambda i: (i, 0)),
            pl.BlockSpec((TC_BLK, 8), lambda i: (i, 0)),
            pl.BlockSpec((1, H), lambda i: (0, 0)),
            pl.BlockSpec((H, H), lambda i: (0, 0)),
        ],
        out_specs=pl.BlockSpec((TC_BLK, 2 * H), lambda i: (i, 0)),
        out_shape=jax.ShapeDtypeStruct((N_PAD, 2 * H), jnp.float32),
    )(sp, q, dinv, b.reshape(1, H), W_next)


def _proj_body(s_ref, q_ref, dinv_ref, b_ref, wc_ref, ab_ref):
    dinv = dinv_ref[:, 0:1]
    h = jax.nn.relu(
        (s_ref[0][:, :H] + s_ref[1][:, :H] + q_ref[:, :H]) * dinv
        + b_ref[...])
    ab_ref[...] = jnp.dot(h, wc_ref[...], preferred_element_type=jnp.float32)


def _tc_proj(sp, q, dinv, b, Wf1):
    return pl.pallas_call(
        _proj_body,
        grid=(TC_GRID,),
        in_specs=[
            pl.BlockSpec((NC, TC_BLK, 2 * H), lambda i: (0, i, 0)),
            pl.BlockSpec((TC_BLK, 2 * H), lambda i: (i, 0)),
            pl.BlockSpec((TC_BLK, 8), lambda i: (i, 0)),
            pl.BlockSpec((1, H), lambda i: (0, 0)),
            pl.BlockSpec((H, 2 * H), lambda i: (0, 0)),
        ],
        out_specs=pl.BlockSpec((TC_BLK, 2 * H), lambda i: (i, 0)),
        out_shape=jax.ShapeDtypeStruct((N_PAD, 2 * H), jnp.float32),
    )(sp, q, dinv, b.reshape(1, H),
      jnp.concatenate([Wf1[:H], Wf1[H:]], axis=1))


MLP_BLK = 1000
MLP_GRID = P // MLP_BLK


def _mlp_body(gs_ref, gt_ref, b1_ref, w2_ref, b2_ref, w3_ref, b3_ref, out_ref):
    z = jax.nn.relu(gs_ref[:, :H] + gt_ref[:, H:] + b1_ref[...])
    z2 = jax.nn.relu(
        jnp.dot(z, w2_ref[...], preferred_element_type=jnp.float32)
        + b2_ref[...])
    out_ref[...] = jnp.dot(z2, w3_ref[...],
                           preferred_element_type=jnp.float32) + b3_ref[...]


def _tc_mlp(gs, gt, bf1, Wf2, bf2, Wf3, bf3):
    return pl.pallas_call(
        _mlp_body,
        grid=(MLP_GRID,),
        in_specs=[
            pl.BlockSpec((MLP_BLK, 2 * H), lambda i: (i, 0)),
            pl.BlockSpec((MLP_BLK, 2 * H), lambda i: (i, 0)),
            pl.BlockSpec((1, H), lambda i: (0, 0)),
            pl.BlockSpec((H, H // 2), lambda i: (0, 0)),
            pl.BlockSpec((1, H // 2), lambda i: (0, 0)),
            pl.BlockSpec((H // 2, 1), lambda i: (0, 0)),
            pl.BlockSpec((1, 1), lambda i: (0, 0)),
        ],
        out_specs=pl.BlockSpec((MLP_BLK, 1), lambda i: (i, 0)),
        out_shape=jax.ShapeDtypeStruct((P, 1), jnp.float32),
    )(gs, gt, bf1.reshape(1, H), Wf2, bf2.reshape(1, H // 2), Wf3,
      bf3.reshape(1, 1))


def kernel(x, edge_index, flow_edges, W_enc, b_enc, W1, b1, W2, b2, W3, b3,
           Wf1, bf1, Wf2, bf2, Wf3, bf3):
    x_pad = jnp.concatenate(
        [x, jnp.zeros((N_PAD - N, F_IN), jnp.float32)], axis=0)

    def pad_flat(idx, total, fill):
        idx = idx.astype(jnp.int32)
        return jnp.concatenate(
            [idx, jnp.full((total - idx.shape[0],), fill, jnp.int32)])

    srcf = pad_flat(edge_index[0], E_PAD, N)
    dstf = pad_flat(edge_index[1], E_PAD, N)
    src32 = srcf.reshape(E_PAD // 128, 128)
    dst32 = dstf.reshape(E_PAD // 128, 128)
    dst32w = dstf.reshape(NW, E_BLOCKS, 128)

    fsrc = pad_flat(flow_edges[0], P_PAD, 0)
    ftgt = pad_flat(flow_edges[1], P_PAD, 0)
    fflat = jnp.concatenate([fsrc, ftgt]).reshape(NW * FB, 128)
    fA = fflat[:NS * NF0].reshape(NS, NF0, 128)
    fB = fflat[NS * NF0:].reshape(NS, NF1, 128)

    zeros128 = jnp.zeros((N_PAD, 2 * H), jnp.float32)
    ones128 = jnp.ones((128, 2 * H), jnp.float32)

    degp = _deg_kernel(dst32w, ones128, zeros128).reshape(NC, N_PAD, 2 * H)

    q1, dinv = _tc_encoder(x_pad, degp, W_enc, b_enc, W1)

    s1 = _edge_scatter_kernel(q1, src32, dst32,
                              zeros128).reshape(NC, N_PAD, 2 * H)
    q2 = _tc_layer(s1, q1, dinv, b1, W2)
    s2 = _edge_scatter_kernel(q2, src32, dst32,
                              zeros128).reshape(NC, N_PAD, 2 * H)
    q3 = _tc_layer(s2, q2, dinv, b2, W3)
    s3 = _edge_scatter_kernel(q3, src32, dst32,
                              zeros128).reshape(NC, N_PAD, 2 * H)

    AB = _tc_proj(s3, q3, dinv, b3, Wf1)
    g = _flow_gather_kernel(AB, fA, fB)
    gs = g[:P_PAD]
    gt = g[P_PAD:2 * P_PAD]
    flows = _tc_mlp(gs[:P], gt[:P], bf1, Wf2, bf2, Wf3, bf3)
    return flows
